# baseline (speedup 1.0000x reference)
import functools

import jax
import jax.numpy as jnp
from jax import lax
from jax.experimental import pallas as pl
from jax.experimental.pallas import tpu as pltpu
from jax.experimental.pallas import tpu_sc as plsc

_CH = 128


@functools.lru_cache(maxsize=None)
def _build_gather(n_rows, d):
    info = plsc.get_sparse_core_info()
    nw = info.num_cores * info.num_subcores
    n_per_w = n_rows // nw
    n_ch = n_per_w // _CH
    mesh = plsc.VectorSubcoreMesh(core_axis_name="c", subcore_axis_name="s")

    @functools.partial(
        pl.kernel,
        out_type=jax.ShapeDtypeStruct((n_rows, d), jnp.float32),
        mesh=mesh,
        compiler_params=pltpu.CompilerParams(use_tc_tiling_on_sc=False),
        scratch_types=[
            pltpu.VMEM((n_ch, _CH), jnp.int32),
            pltpu.VMEM((_CH, d), jnp.float32),
            pltpu.SemaphoreType.DMA,
        ],
    )
    def gather_kernel(idx_hbm, table_hbm, out_hbm, idx_v, buf, sem):
        wid = lax.axis_index("s") * info.num_cores + lax.axis_index("c")
        base = wid * n_ch
        pltpu.sync_copy(idx_hbm.at[pl.ds(base, n_ch)], idx_v)

        def body(j, carry):
            pltpu.async_copy(table_hbm.at[idx_v.at[j]], buf, sem).wait()
            pltpu.sync_copy(buf, out_hbm.at[pl.ds((base + j) * _CH, _CH)])
            return carry

        lax.fori_loop(0, n_ch, body, 0)

    return gather_kernel


def kernel(x, token_table):
    b, t = x.shape
    _, d = token_table.shape
    n = b * t
    idx = x.reshape(n // _CH, _CH).astype(jnp.int32)
    out = _build_gather(n, d)(idx, token_table)
    return out.reshape(b, t, d)


# speedup vs baseline: 1.1109x; 1.1109x over previous
import functools

import jax
import jax.numpy as jnp
from jax import lax
from jax.experimental import pallas as pl
from jax.experimental.pallas import tpu as pltpu
from jax.experimental.pallas import tpu_sc as plsc

_CH = 128
_K = 4


@functools.lru_cache(maxsize=None)
def _build_gather(n_rows, d):
    info = plsc.get_sparse_core_info()
    nw = info.num_cores * info.num_subcores
    n_per_w = n_rows // nw
    n_ch = n_per_w // _CH
    n_g = n_ch // _K
    mesh = plsc.VectorSubcoreMesh(core_axis_name="c", subcore_axis_name="s")

    @functools.partial(
        pl.kernel,
        out_type=jax.ShapeDtypeStruct((n_rows // _CH, _CH, d), jnp.float32),
        mesh=mesh,
        compiler_params=pltpu.CompilerParams(use_tc_tiling_on_sc=False),
        scratch_types=[
            pltpu.VMEM((n_ch, _CH), jnp.int32),
            pltpu.VMEM((2, _K, _CH, d), jnp.float32),
            pltpu.SemaphoreType.DMA,
            pltpu.SemaphoreType.DMA,
        ],
    )
    def gather_kernel(idx_hbm, table_hbm, out_hbm, idx_v, buf, gsem, wsem):
        wid = lax.axis_index("s") * info.num_cores + lax.axis_index("c")
        base = wid * n_ch
        pltpu.sync_copy(idx_hbm.at[pl.ds(base, n_ch)], idx_v)

        def gathers(g, parity):
            return [
                pltpu.make_async_copy(
                    table_hbm.at[idx_v.at[g * _K + b]], buf.at[parity, b], gsem
                )
                for b in range(_K)
            ]

        def write(g, parity):
            return pltpu.make_async_copy(
                buf.at[parity], out_hbm.at[pl.ds(base + g * _K, _K)], wsem
            )

        for c in gathers(0, 0):
            c.start()

        def body(g, carry):
            parity = lax.rem(g, 2)
            for c in gathers(g, parity):
                c.wait()

            @pl.when(g > 0)
            def _():
                write(g - 1, 1 - parity).wait()

            write(g, parity).start()

            @pl.when(g < n_g - 1)
            def _():
                for c in gathers(g + 1, 1 - parity):
                    c.start()

            return carry

        lax.fori_loop(0, n_g, body, 0)
        write(n_g - 1, (n_g - 1) % 2).wait()

    return gather_kernel


def kernel(x, token_table):
    b, t = x.shape
    _, d = token_table.shape
    n = b * t
    idx = x.reshape(n // _CH, _CH).astype(jnp.int32)
    out = _build_gather(n, d)(idx, token_table)
    return out.reshape(b, t, d)
